# Initial kernel scaffold; baseline (speedup 1.0000x reference)
import jax
import jax.numpy as jnp
from jax.experimental import pallas as pl


def kernel(x, edge_index, edge_attr, params):
    raise NotImplementedError("write your pallas kernel here")


# speedup vs baseline: 1.5746x; 1.5746x over previous
import functools

import numpy as np
import jax
import jax.numpy as jnp
from jax import lax
from jax.experimental import pallas as pl
from jax.experimental.pallas import tpu as pltpu
from jax.experimental.pallas import tpu_sc as plsc

N = 10000
E = 160000
EE = 163840
F = 16
W128 = 128
NC = 2
NS = 16
NW = NC * NS
PT = EE // NW
CH = 128
ROWS = PT // CH
NPAD = 10240
NSEG = 2
SEGN = NPAD // NSEG
SEGT = 128
SEGR = SEGN + SEGT
RPS = SEGN // NS
RZB = SEGR // NS

_DIMS = [2, 4, 8, 16]

_mesh = plsc.VectorSubcoreMesh(core_axis_name="c", subcore_axis_name="s")


def _gather_body(tab_hbm, idx_hbm, out_hbm, idx_v, buf_v, sem):
    c = lax.axis_index("c")
    s = lax.axis_index("s")
    wid = s * NC + c
    pltpu.sync_copy(idx_hbm.at[wid], idx_v)

    def chunk(j, carry):
        pltpu.async_copy(tab_hbm.at[idx_v.at[j]], buf_v, sem).wait()
        pltpu.sync_copy(buf_v, out_hbm.at[pl.ds(wid * PT + j * CH, CH)])
        return carry

    lax.fori_loop(0, ROWS, chunk, 0)


def _gather(tab, idx3):
    return pl.kernel(
        _gather_body,
        out_type=jax.ShapeDtypeStruct((EE, W128), jnp.float32),
        mesh=_mesh,
        scratch_types=[
            pltpu.VMEM((ROWS, CH), jnp.int32),
            pltpu.VMEM((CH, W128), jnp.float32),
            pltpu.SemaphoreType.DMA,
        ],
    )(tab, idx3)


def _scatter_body(msg_hbm, idx_hbm, out_hbm, idx_v, idx2_v, buf_v, zb_v,
                  shared):
    c = lax.axis_index("c")
    s = lax.axis_index("s")
    wid = s * NC + c

    def zrow(j, carry):
        def zlane(l, carry2):
            zb_v[j, pl.ds(l * F, F)] = jnp.zeros((F,), jnp.float32)
            return carry2
        return lax.fori_loop(0, W128 // F, zlane, carry)

    lax.fori_loop(0, RZB, zrow, 0)

    pltpu.sync_copy(idx_hbm.at[wid], idx_v)

    for p in range(NSEG):
        base = p * SEGN
        pltpu.sync_copy(zb_v, shared.at[pl.ds(s * RZB, RZB)])

        def remap(j, carry):
            def grp(g, carry2):
                t = idx_v[j, pl.ds(g * F, F)] - base
                ok = (t >= 0) & (t < SEGN)
                tr = (t & (SEGT - 1)) + SEGN
                idx2_v[j, pl.ds(g * F, F)] = jnp.where(ok, t, tr)
                return carry2
            return lax.fori_loop(0, CH // F, grp, carry)

        lax.fori_loop(0, ROWS, remap, 0)
        plsc.subcore_barrier()

        def chunk(j, carry):
            pltpu.sync_copy(msg_hbm.at[pl.ds(wid * PT + j * CH, CH)], buf_v)
            pltpu.sync_copy(buf_v, shared.at[idx2_v.at[j]], add=True)
            return carry

        lax.fori_loop(0, ROWS, chunk, 0)
        plsc.subcore_barrier()
        pltpu.sync_copy(shared.at[pl.ds(s * RPS, RPS)],
                        out_hbm.at[c, p, pl.ds(s * RPS, RPS)])
        plsc.subcore_barrier()


def _scatter(msg, idx3):
    return pl.kernel(
        _scatter_body,
        out_type=jax.ShapeDtypeStruct((NC, NSEG, SEGN, W128), jnp.float32),
        mesh=_mesh,
        scratch_types=[
            pltpu.VMEM((ROWS, CH), jnp.int32),
            pltpu.VMEM((ROWS, CH), jnp.int32),
            pltpu.VMEM((CH, W128), jnp.float32),
            pltpu.VMEM((RZB, W128), jnp.float32),
            pltpu.VMEM_SHARED((SEGR, W128), jnp.float32),
        ],
    )(msg, idx3)


_BLK = 4096


def _msg_body(ea, hs, kw0, kb0, kw1, kb1, kw2, kb2, kw3, kb3, pm, rm, out):
    f32 = jnp.float32
    e = ea[...]
    e = jnp.maximum(jnp.dot(e, kw0[...], preferred_element_type=f32) + kb0[...], 0.0)
    e = jnp.maximum(jnp.dot(e, kw1[...], preferred_element_type=f32) + kb1[...], 0.0)
    e = jnp.maximum(jnp.dot(e, kw2[...], preferred_element_type=f32) + kb2[...], 0.0)
    w = jnp.dot(e, kw3[...], preferred_element_type=f32) + kb3[...]
    hx = jnp.dot(hs[...], pm[...], preferred_element_type=f32)
    out[...] = jnp.dot(w * hx, rm[...], preferred_element_type=f32)


def _msg(edge_attr, hsrc, kw, kb, pm, rm):
    k = kw[3].shape[1]
    full = lambda shape: pl.BlockSpec(shape, lambda i: tuple(0 for _ in shape))
    grid = EE // _BLK
    return pl.pallas_call(
        _msg_body,
        grid=(grid,),
        in_specs=[
            pl.BlockSpec((_BLK, 4), lambda i: (i, 0)),
            pl.BlockSpec((_BLK, W128), lambda i: (i, 0)),
            full((4, 30)), full((1, 30)),
            full((30, 60)), full((1, 60)),
            full((60, 30)), full((1, 30)),
            full((30, k)), full((1, k)),
            full((W128, k)), full((k, W128)),
        ],
        out_specs=pl.BlockSpec((_BLK, W128), lambda i: (i, 0)),
        out_shape=jax.ShapeDtypeStruct((EE, W128), jnp.float32),
    )(edge_attr, hsrc, kw[0], kb[0], kw[1], kb[1], kw[2], kb[2], kw[3], kb[3],
      pm, rm)


def _update_body(agg, h, root, bias, out):
    a = agg[0] + agg[1]
    out[...] = jnp.maximum(
        a + jnp.dot(h[:, :F], root[...], preferred_element_type=jnp.float32)
        + bias[...], 0.0)


def _update(agg, h, root, bias):
    return pl.pallas_call(
        _update_body,
        out_shape=jax.ShapeDtypeStruct((NPAD, W128), jnp.float32),
    )(agg, h, root, bias)


def _final_body(agg, h, root, bias, w0, b0, w1, b1, w2, b2, out):
    f32 = jnp.float32
    a = agg[0, :N, :] + agg[1, :N, :]
    h3 = jnp.maximum(
        a + jnp.dot(h[:N, :F], root[...], preferred_element_type=f32)
        + bias[...], 0.0)
    s = jnp.sum(h3, axis=0, keepdims=True)[:, :F]
    s = jnp.maximum(jnp.dot(s, w0[...], preferred_element_type=f32) + b0[...], 0.0)
    s = jnp.maximum(jnp.dot(s, w1[...], preferred_element_type=f32) + b1[...], 0.0)
    out[...] = jnp.dot(s, w2[...], preferred_element_type=f32) + b2[...]


def _final(agg, h, root, bias, mw, mb):
    return pl.pallas_call(
        _final_body,
        out_shape=jax.ShapeDtypeStruct((1, 1), jnp.float32),
    )(agg, h, root, bias, mw[0], mb[0], mw[1], mb[1], mw[2], mb[2])


@functools.lru_cache(maxsize=None)
def _expand_reduce(fin, fout):
    k = fin * fout
    pm = np.zeros((W128, k), np.float32)
    rm = np.zeros((k, W128), np.float32)
    for i in range(fin):
        for o in range(fout):
            pm[i, i * fout + o] = 1.0
            rm[i * fout + o, o] = 1.0
    return jnp.asarray(pm), jnp.asarray(rm)


def kernel(x, edge_index, edge_attr, params):
    pad_e = EE - E
    srcp = jnp.concatenate(
        [edge_index[0], jnp.zeros((pad_e,), jnp.int32)])
    dead = jnp.full((pad_e,), NPAD, jnp.int32)
    dstp = jnp.concatenate([edge_index[1], dead])
    src3 = srcp.reshape(NW, ROWS, CH)
    dst3 = dstp.reshape(NW, ROWS, CH)
    ea = jnp.pad(edge_attr, ((0, pad_e), (0, 0)))

    h = jnp.zeros((NPAD, W128), jnp.float32).at[:N, :x.shape[1]].set(x)
    hsrc = _gather(h, src3)
    for l in range(3):
        p = params['ecc%d' % l]
        fin, fout = _DIMS[l], _DIMS[l + 1]
        pm, rm = _expand_reduce(fin, fout)
        kb = [b.reshape(1, -1) for b in p['kb']]
        msg = _msg(ea, hsrc, p['kw'], kb, pm, rm)
        agg = _scatter(msg, dst3).reshape(NC, NPAD, W128)
        root = jnp.zeros((F, W128), jnp.float32).at[:fin, :fout].set(p['root'])
        bias = jnp.zeros((1, W128), jnp.float32).at[0, :fout].set(p['bias'])
        if l < 2:
            h = _update(agg, h, root, bias)
            hsrc = _gather(h, src3)
        else:
            out = _final(agg, h, root, bias,
                         params['mlp']['w'],
                         [b.reshape(1, -1) for b in params['mlp']['b']])
    return out.reshape(1)


# speedup vs baseline: 1.9141x; 1.2156x over previous
import functools

import numpy as np
import jax
import jax.numpy as jnp
from jax import lax
from jax.experimental import pallas as pl
from jax.experimental.pallas import tpu as pltpu
from jax.experimental.pallas import tpu_sc as plsc

N = 10000
E = 160000
EE = 163840
F = 16
W128 = 128
NC = 2
NS = 16
NW = NC * NS
PT = EE // NW
CH = 128
ROWS = PT // CH
NPAD = 10240
NSEG = 2
SEGN = NPAD // NSEG
SEGT = 128
SEGR = SEGN + SEGT
RPS = SEGN // NS
RZB = SEGR // NS

_DIMS = [2, 4, 8, 16]

_mesh = plsc.VectorSubcoreMesh(core_axis_name="c", subcore_axis_name="s")


def _gather_body(tab_hbm, idx_hbm, out_hbm, idx_v, buf0_v, buf1_v, sem0,
                 sem1):
    c = lax.axis_index("c")
    s = lax.axis_index("s")
    wid = s * NC + c
    pltpu.sync_copy(idx_hbm.at[wid], idx_v)

    bufs = [buf0_v, buf1_v]
    sems = [sem0, sem1]
    pend = [None, None]
    pend[0] = pltpu.async_copy(tab_hbm.at[idx_v.at[0]], bufs[0], sems[0])
    for j in range(ROWS):
        if j + 1 < ROWS:
            pend[(j + 1) % 2] = pltpu.async_copy(
                tab_hbm.at[idx_v.at[j + 1]], bufs[(j + 1) % 2],
                sems[(j + 1) % 2])
        pend[j % 2].wait()
        pltpu.sync_copy(bufs[j % 2], out_hbm.at[pl.ds(wid * PT + j * CH, CH)])


def _gather(tab, idx3):
    return pl.kernel(
        _gather_body,
        out_type=jax.ShapeDtypeStruct((EE, W128), jnp.float32),
        mesh=_mesh,
        scratch_types=[
            pltpu.VMEM((ROWS, CH), jnp.int32),
            pltpu.VMEM((CH, W128), jnp.float32),
            pltpu.VMEM((CH, W128), jnp.float32),
            pltpu.SemaphoreType.DMA,
            pltpu.SemaphoreType.DMA,
        ],
    )(tab, idx3)


def _scatter_body(msg_hbm, idx_hbm, out_hbm, idx_v, idx2_v, buf0_v, buf1_v,
                  zb_v, shared, sem0, sem1):
    c = lax.axis_index("c")
    s = lax.axis_index("s")
    wid = s * NC + c

    def zrow(j, carry):
        def zlane(l, carry2):
            zb_v[j, pl.ds(l * F, F)] = jnp.zeros((F,), jnp.float32)
            return carry2
        return lax.fori_loop(0, W128 // F, zlane, carry)

    lax.fori_loop(0, RZB, zrow, 0)

    pltpu.sync_copy(idx_hbm.at[wid], idx_v)

    bufs = [buf0_v, buf1_v]
    sems = [sem0, sem1]
    for p in range(NSEG):
        base = p * SEGN
        pltpu.sync_copy(zb_v, shared.at[pl.ds(s * RZB, RZB)])

        def remap(j, carry):
            def grp(g, carry2):
                t = idx_v[j, pl.ds(g * F, F)] - base
                ok = (t >= 0) & (t < SEGN)
                tr = (t & (SEGT - 1)) + SEGN
                idx2_v[j, pl.ds(g * F, F)] = jnp.where(ok, t, tr)
                return carry2
            return lax.fori_loop(0, CH // F, grp, carry)

        lax.fori_loop(0, ROWS, remap, 0)
        plsc.subcore_barrier()

        pend = [None, None]
        pend[0] = pltpu.async_copy(
            msg_hbm.at[pl.ds(wid * PT, CH)], bufs[0], sems[0])
        for j in range(ROWS):
            if j + 1 < ROWS:
                pend[(j + 1) % 2] = pltpu.async_copy(
                    msg_hbm.at[pl.ds(wid * PT + (j + 1) * CH, CH)],
                    bufs[(j + 1) % 2], sems[(j + 1) % 2])
            pend[j % 2].wait()
            pltpu.sync_copy(bufs[j % 2], shared.at[idx2_v.at[j]], add=True)

        plsc.subcore_barrier()
        pltpu.sync_copy(shared.at[pl.ds(s * RPS, RPS)],
                        out_hbm.at[c, p, pl.ds(s * RPS, RPS)])
        plsc.subcore_barrier()


def _scatter(msg, idx3):
    return pl.kernel(
        _scatter_body,
        out_type=jax.ShapeDtypeStruct((NC, NSEG, SEGN, W128), jnp.float32),
        mesh=_mesh,
        scratch_types=[
            pltpu.VMEM((ROWS, CH), jnp.int32),
            pltpu.VMEM((ROWS, CH), jnp.int32),
            pltpu.VMEM((CH, W128), jnp.float32),
            pltpu.VMEM((CH, W128), jnp.float32),
            pltpu.VMEM((RZB, W128), jnp.float32),
            pltpu.VMEM_SHARED((SEGR, W128), jnp.float32),
            pltpu.SemaphoreType.DMA,
            pltpu.SemaphoreType.DMA,
        ],
    )(msg, idx3)


_BLK = 4096


def _msg_body(ea, hs, kw0, kb0, kw1, kb1, kw2, kb2, kw3, kb3, pm, rm, out):
    f32 = jnp.float32
    e = ea[...]
    e = jnp.maximum(jnp.dot(e, kw0[...], preferred_element_type=f32) + kb0[...], 0.0)
    e = jnp.maximum(jnp.dot(e, kw1[...], preferred_element_type=f32) + kb1[...], 0.0)
    e = jnp.maximum(jnp.dot(e, kw2[...], preferred_element_type=f32) + kb2[...], 0.0)
    w = jnp.dot(e, kw3[...], preferred_element_type=f32) + kb3[...]
    hx = jnp.dot(hs[...], pm[...], preferred_element_type=f32)
    out[...] = jnp.dot(w * hx, rm[...], preferred_element_type=f32)


def _msg(edge_attr, hsrc, kw, kb, pm, rm):
    k = kw[3].shape[1]
    full = lambda shape: pl.BlockSpec(shape, lambda i: tuple(0 for _ in shape))
    grid = EE // _BLK
    return pl.pallas_call(
        _msg_body,
        grid=(grid,),
        in_specs=[
            pl.BlockSpec((_BLK, 4), lambda i: (i, 0)),
            pl.BlockSpec((_BLK, W128), lambda i: (i, 0)),
            full((4, 30)), full((1, 30)),
            full((30, 60)), full((1, 60)),
            full((60, 30)), full((1, 30)),
            full((30, k)), full((1, k)),
            full((W128, k)), full((k, W128)),
        ],
        out_specs=pl.BlockSpec((_BLK, W128), lambda i: (i, 0)),
        out_shape=jax.ShapeDtypeStruct((EE, W128), jnp.float32),
    )(edge_attr, hsrc, kw[0], kb[0], kw[1], kb[1], kw[2], kb[2], kw[3], kb[3],
      pm, rm)


def _update_body(agg, h, root, bias, out):
    a = agg[0] + agg[1]
    out[...] = jnp.maximum(
        a + jnp.dot(h[:, :F], root[...], preferred_element_type=jnp.float32)
        + bias[...], 0.0)


def _update(agg, h, root, bias):
    return pl.pallas_call(
        _update_body,
        out_shape=jax.ShapeDtypeStruct((NPAD, W128), jnp.float32),
    )(agg, h, root, bias)


def _final_body(agg, h, root, bias, w0, b0, w1, b1, w2, b2, out):
    f32 = jnp.float32
    a = agg[0, :N, :] + agg[1, :N, :]
    h3 = jnp.maximum(
        a + jnp.dot(h[:N, :F], root[...], preferred_element_type=f32)
        + bias[...], 0.0)
    s = jnp.sum(h3, axis=0, keepdims=True)[:, :F]
    s = jnp.maximum(jnp.dot(s, w0[...], preferred_element_type=f32) + b0[...], 0.0)
    s = jnp.maximum(jnp.dot(s, w1[...], preferred_element_type=f32) + b1[...], 0.0)
    out[...] = jnp.dot(s, w2[...], preferred_element_type=f32) + b2[...]


def _final(agg, h, root, bias, mw, mb):
    return pl.pallas_call(
        _final_body,
        out_shape=jax.ShapeDtypeStruct((1, 1), jnp.float32),
    )(agg, h, root, bias, mw[0], mb[0], mw[1], mb[1], mw[2], mb[2])


@functools.lru_cache(maxsize=None)
def _expand_reduce(fin, fout):
    k = fin * fout
    pm = np.zeros((W128, k), np.float32)
    rm = np.zeros((k, W128), np.float32)
    for i in range(fin):
        for o in range(fout):
            pm[i, i * fout + o] = 1.0
            rm[i * fout + o, o] = 1.0
    return jnp.asarray(pm), jnp.asarray(rm)


def kernel(x, edge_index, edge_attr, params):
    pad_e = EE - E
    srcp = jnp.concatenate(
        [edge_index[0], jnp.zeros((pad_e,), jnp.int32)])
    dead = jnp.full((pad_e,), NPAD, jnp.int32)
    dstp = jnp.concatenate([edge_index[1], dead])
    src3 = srcp.reshape(NW, ROWS, CH)
    dst3 = dstp.reshape(NW, ROWS, CH)
    ea = jnp.pad(edge_attr, ((0, pad_e), (0, 0)))

    h = jnp.zeros((NPAD, W128), jnp.float32).at[:N, :x.shape[1]].set(x)
    hsrc = _gather(h, src3)
    for l in range(3):
        p = params['ecc%d' % l]
        fin, fout = _DIMS[l], _DIMS[l + 1]
        pm, rm = _expand_reduce(fin, fout)
        kb = [b.reshape(1, -1) for b in p['kb']]
        msg = _msg(ea, hsrc, p['kw'], kb, pm, rm)
        agg = _scatter(msg, dst3).reshape(NC, NPAD, W128)
        root = jnp.zeros((F, W128), jnp.float32).at[:fin, :fout].set(p['root'])
        bias = jnp.zeros((1, W128), jnp.float32).at[0, :fout].set(p['bias'])
        if l < 2:
            h = _update(agg, h, root, bias)
            hsrc = _gather(h, src3)
        else:
            out = _final(agg, h, root, bias,
                         params['mlp']['w'],
                         [b.reshape(1, -1) for b in params['mlp']['b']])
    return out.reshape(1)
